# Initial kernel scaffold; baseline (speedup 1.0000x reference)
import jax
import jax.numpy as jnp
from jax.experimental import pallas as pl


def kernel(y, x, colors, height, width):
    raise NotImplementedError("write your pallas kernel here")


# speedup vs baseline: 1.8433x; 1.8433x over previous
import functools

import jax
import jax.numpy as jnp
from jax import lax
from jax.experimental import pallas as pl
from jax.experimental.pallas import tpu as pltpu
from jax.experimental.pallas import tpu_sc as plsc

_H = 1024
_W = 1024
_N = 4_000_000

_NC = 2
_NS = 16
_NW = _NC * _NS
_BAND_ROWS = _H // _NW
_BAND_PIX = _BAND_ROWS * _W
_BAND_F32 = _BAND_PIX * 3

_CHUNK = 2000
_NCHUNK = _N // _CHUNK
_NVEC = _CHUNK // 16

_BIG = 0x7FFFFFFF


def _body(y_hbm, x_hbm, col_hbm, out_hbm, y_buf, x_buf, c_buf, band_buf, k_buf):
    wid = lax.axis_index("s") * _NC + lax.axis_index("c")
    iota = lax.iota(jnp.int32, 16)

    def zero_body(i, _):
        band_buf[pl.ds(i * 16, 16)] = jnp.zeros((16,), jnp.float32)
        return 0

    lax.fori_loop(0, _BAND_F32 // 16, zero_body, 0)

    def vec_body(v, _):
        yv = y_buf[pl.ds(v * 16, 16)]
        m = lax.shift_right_logical(yv, 5) == wid

        if True:
            xv = x_buf[pl.ds(v * 16, 16)]
            pix = (yv & (_BAND_ROWS - 1)) * _W + xv
            key = jnp.where(m, lax.shift_left(pix, 4) | iota, _BIG)
            ks = lax.sort(key)
            k_buf[...] = ks
            nxt = plsc.load_gather(k_buf, [jnp.minimum(iota + 1, 15)])
            pix_s = lax.shift_right_logical(ks, 4)
            is_last = (ks != _BIG) & ((pix_s != lax.shift_right_logical(nxt, 4))
                                      | (iota == 15))
            lane_s = ks & 15
            pix_c = jnp.where(is_last, pix_s, 0)
            src = (v * 16 + lane_s) * 3
            for ch in range(3):
                col = plsc.load_gather(c_buf, [src + ch], mask=is_last)
                plsc.store_scatter(band_buf, [pix_c * 3 + ch], col, mask=is_last)

        return 0

    def chunk_body(c, _):
        base = c * _CHUNK
        pltpu.sync_copy(y_hbm.at[pl.ds(base, _CHUNK)], y_buf)
        pltpu.sync_copy(x_hbm.at[pl.ds(base, _CHUNK)], x_buf)
        pltpu.sync_copy(col_hbm.at[pl.ds(base * 3, _CHUNK * 3)], c_buf)
        lax.fori_loop(0, _NVEC, vec_body, 0)
        return 0

    lax.fori_loop(0, _NCHUNK, chunk_body, 0)

    pltpu.sync_copy(band_buf, out_hbm.at[pl.ds(wid * _BAND_F32, _BAND_F32)])


@jax.jit
def _project(y, x, colors_flat):
    mesh = plsc.VectorSubcoreMesh(core_axis_name="c", subcore_axis_name="s",
                                  num_cores=_NC, num_subcores=_NS)
    f = pl.kernel(
        _body,
        out_type=jax.ShapeDtypeStruct((_H * _W * 3,), jnp.float32),
        mesh=mesh,
        compiler_params=pltpu.CompilerParams(needs_layout_passes=False),
        scratch_types=[
            pltpu.VMEM((_CHUNK,), jnp.int32),
            pltpu.VMEM((_CHUNK,), jnp.int32),
            pltpu.VMEM((_CHUNK * 3,), jnp.float32),
            pltpu.VMEM((_BAND_F32,), jnp.float32),
            pltpu.VMEM((16,), jnp.int32),
        ],
    )
    return f(y, x, colors_flat)


def kernel(y, x, colors, height, width):
    zero_off = (jnp.asarray(height) - _H) + (jnp.asarray(width) - _W)
    y = (y + zero_off.astype(y.dtype)).astype(jnp.int32)
    x = (x + zero_off.astype(x.dtype)).astype(jnp.int32)
    out_flat = _project(y, x, colors.reshape(-1))
    return out_flat.reshape(_H, _W, 3)


# speedup vs baseline: 1.8877x; 1.0241x over previous
import jax
import jax.numpy as jnp
from jax import lax
from jax.experimental import pallas as pl
from jax.experimental.pallas import tpu as pltpu
from jax.experimental.pallas import tpu_sc as plsc

_H = 1024
_W = 1024
_N = 4_000_000

_NC = 2
_NS = 16
_NW = _NC * _NS
_BAND_ROWS = _H // _NW
_BAND_F32 = _BAND_ROWS * _W * 3

_CHUNK = 2000
_NCHUNK = _N // _CHUNK
_NVEC = _CHUNK // 16

_TAG = 4096

_BIG = 0x7FFFFFFF


def _body(y_hbm, x_hbm, col_hbm, out_hbm,
          y0, x0, c0, y1, x1, c1, band_buf, k_buf, tag, sem0, sem1):
    wid = lax.axis_index("s") * _NC + lax.axis_index("c")
    iota = lax.iota(jnp.int32, 16)

    def zero_body(i, _):
        band_buf[pl.ds(i * 16, 16)] = jnp.zeros((16,), jnp.float32)
        return 0

    lax.fori_loop(0, _BAND_F32 // 16, zero_body, 0)

    def issue(c, yb, xb, cb, sem):
        base = c * _CHUNK
        pltpu.async_copy(y_hbm.at[pl.ds(base, _CHUNK)], yb, sem)
        pltpu.async_copy(x_hbm.at[pl.ds(base, _CHUNK)], xb, sem)
        pltpu.async_copy(col_hbm.at[pl.ds(base * 3, _CHUNK * 3)], cb, sem)

    def drain(yb, xb, cb, sem):
        pltpu.make_async_copy(y_hbm.at[pl.ds(0, _CHUNK)], yb, sem).wait()
        pltpu.make_async_copy(x_hbm.at[pl.ds(0, _CHUNK)], xb, sem).wait()
        pltpu.make_async_copy(col_hbm.at[pl.ds(0, _CHUNK * 3)], cb, sem).wait()

    def compute(yb, xb, cb):
        def vec_body(v, _):
            yv = yb[pl.ds(v * 16, 16)]
            m = lax.shift_right_logical(yv, 5) == wid

            @pl.when(jnp.any(m))
            def _():
                xv = xb[pl.ds(v * 16, 16)]
                pix = (yv & (_BAND_ROWS - 1)) * _W + xv
                pixc = jnp.where(m, pix, 0)
                tok = lax.shift_left(v.astype(jnp.int32), 4) | iota
                slot = pixc & (_TAG - 1)
                plsc.store_scatter(tag, [slot], tok, mask=m)
                got = plsc.load_gather(tag, [slot], mask=m)
                clash = jnp.any(m & (got != tok))

                @pl.when(jnp.logical_not(clash))
                def _():
                    src = (v * 16 + iota) * 3
                    for ch in range(3):
                        col = plsc.load_gather(cb, [src + ch], mask=m)
                        plsc.store_scatter(band_buf, [pixc * 3 + ch], col,
                                           mask=m)

                @pl.when(clash)
                def _():
                    key = jnp.where(m, lax.shift_left(pix, 4) | iota, _BIG)
                    ks = lax.sort(key)
                    k_buf[...] = ks
                    nxt = plsc.load_gather(k_buf, [jnp.minimum(iota + 1, 15)])
                    pix_s = lax.shift_right_logical(ks, 4)
                    is_last = (ks != _BIG) & (
                        (pix_s != lax.shift_right_logical(nxt, 4))
                        | (iota == 15))
                    lane_s = ks & 15
                    pix_w = jnp.where(is_last, pix_s, 0)
                    src = (v * 16 + lane_s) * 3
                    for ch in range(3):
                        col = plsc.load_gather(cb, [src + ch], mask=is_last)
                        plsc.store_scatter(band_buf, [pix_w * 3 + ch], col,
                                           mask=is_last)

            return 0

        lax.fori_loop(0, _NVEC, vec_body, 0)

    issue(0, y0, x0, c0, sem0)

    def pair_body(i, _):
        drain(y0, x0, c0, sem0)
        issue(2 * i + 1, y1, x1, c1, sem1)
        compute(y0, x0, c0)
        drain(y1, x1, c1, sem1)

        @pl.when(2 * i + 2 < _NCHUNK)
        def _():
            issue(2 * i + 2, y0, x0, c0, sem0)

        compute(y1, x1, c1)
        return 0

    lax.fori_loop(0, _NCHUNK // 2, pair_body, 0)

    pltpu.sync_copy(band_buf, out_hbm.at[pl.ds(wid * _BAND_F32, _BAND_F32)])


@jax.jit
def _project(y, x, colors_flat):
    mesh = plsc.VectorSubcoreMesh(core_axis_name="c", subcore_axis_name="s",
                                  num_cores=_NC, num_subcores=_NS)
    f = pl.kernel(
        _body,
        out_type=jax.ShapeDtypeStruct((_H * _W * 3,), jnp.float32),
        mesh=mesh,
        compiler_params=pltpu.CompilerParams(needs_layout_passes=False),
        scratch_types=[
            pltpu.VMEM((_CHUNK,), jnp.int32),
            pltpu.VMEM((_CHUNK,), jnp.int32),
            pltpu.VMEM((_CHUNK * 3,), jnp.float32),
            pltpu.VMEM((_CHUNK,), jnp.int32),
            pltpu.VMEM((_CHUNK,), jnp.int32),
            pltpu.VMEM((_CHUNK * 3,), jnp.float32),
            pltpu.VMEM((_BAND_F32,), jnp.float32),
            pltpu.VMEM((16,), jnp.int32),
            pltpu.VMEM((_TAG,), jnp.int32),
            pltpu.SemaphoreType.DMA,
            pltpu.SemaphoreType.DMA,
        ],
    )
    return f(y, x, colors_flat)


def kernel(y, x, colors, height, width):
    zero_off = (jnp.asarray(height) - _H) + (jnp.asarray(width) - _W)
    y = (y + zero_off.astype(y.dtype)).astype(jnp.int32)
    x = (x + zero_off.astype(x.dtype)).astype(jnp.int32)
    out_flat = _project(y, x, colors.reshape(-1))
    return out_flat.reshape(_H, _W, 3)


# speedup vs baseline: 3.5138x; 1.8614x over previous
import jax
import jax.numpy as jnp
import numpy as np
from jax import lax
from jax.experimental import pallas as pl
from jax.experimental.pallas import tpu as pltpu
from jax.experimental.pallas import tpu_sc as plsc

_H = 1024
_W = 1024
_N = 4_000_000
_PAD = 8

_NC = 2
_NS = 16
_NW = _NC * _NS

_HALF = _N // _NC
_B1_ROWS = _H // _NS
_B1_PIX = _B1_ROWS * _W
_CHUNK = 4000
_NCHUNK = _HALF // _CHUNK
_NVEC = _CHUNK // 16

_B2_ROWS = _H // _NW
_B2_PIX = _B2_ROWS * _W
_WIN = 2048
_NWIN = _B2_PIX // _WIN


def _scan_body(y_hbm, x_hbm, w_hbm, y0, x0, y1, x1, wband, sem0, sem1):
    cid = lax.axis_index("c")
    tid = lax.axis_index("s")
    iota = lax.iota(jnp.int32, 16)
    empty = iota & (_PAD - 1)
    lane_bit = lax.convert_element_type(
        lax.bitcast_convert_type(lax.shift_left(iota + 127, 23), jnp.float32),
        jnp.int32)
    lane_thr = lane_bit + lane_bit
    zeros16 = iota & 0
    pt_base = cid * _HALF

    def init_body(i, _):
        wband[pl.ds(i * 16, 16)] = empty
        return 0

    lax.fori_loop(0, _B1_PIX // 16, init_body, 0)

    def issue(c, yb, xb, sem):
        base = pt_base + c * _CHUNK
        pltpu.async_copy(y_hbm.at[pl.ds(base, _CHUNK)], yb, sem)
        pltpu.async_copy(x_hbm.at[pl.ds(base, _CHUNK)], xb, sem)

    def drain(yb, xb, sem):
        pltpu.make_async_copy(y_hbm.at[pl.ds(0, _CHUNK)], yb, sem).wait()
        pltpu.make_async_copy(x_hbm.at[pl.ds(0, _CHUNK)], xb, sem).wait()

    def compute(c, yb, xb):
        idx_base = pt_base + c * _CHUNK + _PAD

        def vec_body(v, _):
            yv = yb[pl.ds(v * 16, 16)]
            xv = xb[pl.ds(v * 16, 16)]
            m = lax.shift_right_logical(yv, 6) == tid
            pix = (yv & (_B1_ROWS - 1)) * _W + xv
            pixc = jnp.where(m, pix, 0)
            plsc.store_scatter(wband, [pixc], zeros16, mask=m)
            plsc.addupdate_scatter(wband, [pixc], lane_bit, mask=m)
            got = plsc.load_gather(wband, [pixc], mask=m)
            keep = m & (got < lane_thr)
            val = (idx_base + v * 16) + iota
            plsc.store_scatter(wband, [pixc], val, mask=keep)
            return 0

        lax.fori_loop(0, _NVEC, vec_body, 0)

    issue(0, y0, x0, sem0)

    def pair_body(i, _):
        drain(y0, x0, sem0)
        issue(2 * i + 1, y1, x1, sem1)
        compute(2 * i, y0, x0)
        drain(y1, x1, sem1)

        @pl.when(2 * i + 2 < _NCHUNK)
        def _():
            issue(2 * i + 2, y0, x0, sem0)

        compute(2 * i + 1, y1, x1)
        return 0

    lax.fori_loop(0, _NCHUNK // 2, pair_body, 0)

    out_off = cid * (_H * _W) + tid * _B1_PIX
    pltpu.sync_copy(wband, w_hbm.at[pl.ds(out_off, _B1_PIX)])


def _merge_body(w_hbm, col_hbm, out_hbm, w0b, w1b, idxb, rows, sem):
    wid = lax.axis_index("s") * _NC + lax.axis_index("c")
    band_px = wid * _B2_PIX

    def win_body(g, _):
        px = band_px + g * _WIN
        pltpu.sync_copy(w_hbm.at[pl.ds(px, _WIN)], w0b)
        pltpu.sync_copy(w_hbm.at[pl.ds(_H * _W + px, _WIN)], w1b)

        def vec_body(v, _):
            a = w0b[pl.ds(v * 16, 16)]
            b = w1b[pl.ds(v * 16, 16)]
            m3 = jnp.maximum(a, b) * 3
            r = lax.div(v, 8)
            cix = lax.rem(v, 8) * 16
            idxb[0, r, pl.ds(cix, 16)] = m3
            idxb[1, r, pl.ds(cix, 16)] = m3 + 1
            idxb[2, r, pl.ds(cix, 16)] = m3 + 2
            return 0

        lax.fori_loop(0, _WIN // 16, vec_body, 0)
        for ch in range(3):
            for j in range(_WIN // 128):
                pltpu.async_copy(col_hbm.at[idxb.at[ch, j]],
                                 rows.at[ch, pl.ds(j * 128, 128)], sem)
        for _ in range(3 * (_WIN // 128)):
            pltpu.make_async_copy(col_hbm.at[idxb.at[0, 0]],
                                  rows.at[0, pl.ds(0, 128)], sem).wait()
        for ch in range(3):
            pltpu.sync_copy(rows.at[ch],
                            out_hbm.at[pl.ds(ch * (_H * _W) + px, _WIN)])
        return 0

    lax.fori_loop(0, _NWIN, win_body, 0)


def _mesh():
    return plsc.VectorSubcoreMesh(core_axis_name="c", subcore_axis_name="s",
                                  num_cores=_NC, num_subcores=_NS)


@jax.jit
def _scan(y, x):
    mesh = _mesh()
    scan = pl.kernel(
        _scan_body,
        out_type=jax.ShapeDtypeStruct((_NC * _H * _W,), jnp.int32),
        mesh=mesh,
        compiler_params=pltpu.CompilerParams(needs_layout_passes=False),
        scratch_types=[
            pltpu.VMEM((_CHUNK,), jnp.int32),
            pltpu.VMEM((_CHUNK,), jnp.int32),
            pltpu.VMEM((_CHUNK,), jnp.int32),
            pltpu.VMEM((_CHUNK,), jnp.int32),
            pltpu.VMEM((_B1_PIX,), jnp.int32),
            pltpu.SemaphoreType.DMA,
            pltpu.SemaphoreType.DMA,
        ],
    )
    return scan(y, x)


@jax.jit
def _merge(w, colors_pad):
    mesh = _mesh()
    merge = pl.kernel(
        _merge_body,
        out_type=jax.ShapeDtypeStruct((3 * _H * _W,), jnp.float32),
        mesh=mesh,
        compiler_params=pltpu.CompilerParams(needs_layout_passes=False,
                                             use_tc_tiling_on_sc=False),
        scratch_types=[
            pltpu.VMEM((_WIN,), jnp.int32),
            pltpu.VMEM((_WIN,), jnp.int32),
            pltpu.VMEM((3, _WIN // 128, 128), jnp.int32),
            pltpu.VMEM((3, _WIN), jnp.float32),
            pltpu.SemaphoreType.DMA,
        ],
    )
    return merge(w, colors_pad)


def kernel(y, x, colors, height, width):
    zero_off = (jnp.asarray(height) - _H) + (jnp.asarray(width) - _W)
    y = (y + zero_off.astype(y.dtype)).astype(jnp.int32)
    x = (x + zero_off.astype(x.dtype)).astype(jnp.int32)
    colors_pad = jnp.concatenate(
        [jnp.zeros((_PAD * 3,), colors.dtype), colors.reshape(-1)])
    out = _merge(_scan(y, x), colors_pad)
    return out.reshape(3, _H, _W).transpose(1, 2, 0)


# speedup vs baseline: 3.5145x; 1.0002x over previous
import jax
import jax.numpy as jnp
import numpy as np
from jax import lax
from jax.experimental import pallas as pl
from jax.experimental.pallas import tpu as pltpu
from jax.experimental.pallas import tpu_sc as plsc

_H = 1024
_W = 1024
_N = 4_000_000
_PAD = 8

_NC = 2
_NS = 16
_NW = _NC * _NS

_HALF = _N // _NC
_B1_ROWS = _H // _NS
_B1_PIX = _B1_ROWS * _W
_CHUNK = 4000
_NCHUNK = _HALF // _CHUNK
_NVEC = _CHUNK // 16

_B2_ROWS = _H // _NW
_B2_PIX = _B2_ROWS * _W
_WIN = 2048
_NWIN = _B2_PIX // _WIN


def _scan_body(y_hbm, x_hbm, w_hbm, y0, x0, y1, x1, wband, sem0, sem1):
    cid = lax.axis_index("c")
    tid = lax.axis_index("s")
    iota = lax.iota(jnp.int32, 16)
    empty = iota & (_PAD - 1)
    lane_bit = lax.convert_element_type(
        lax.bitcast_convert_type(lax.shift_left(iota + 127, 23), jnp.float32),
        jnp.int32)
    lane_thr = lane_bit + lane_bit
    zeros16 = iota & 0
    pt_base = cid * _HALF

    def init_body(i, _):
        wband[pl.ds(i * 16, 16)] = empty
        return 0

    lax.fori_loop(0, _B1_PIX // 16, init_body, 0)

    def issue(c, yb, xb, sem):
        base = pt_base + c * _CHUNK
        pltpu.async_copy(y_hbm.at[pl.ds(base, _CHUNK)], yb, sem)
        pltpu.async_copy(x_hbm.at[pl.ds(base, _CHUNK)], xb, sem)

    def drain(yb, xb, sem):
        pltpu.make_async_copy(y_hbm.at[pl.ds(0, _CHUNK)], yb, sem).wait()
        pltpu.make_async_copy(x_hbm.at[pl.ds(0, _CHUNK)], xb, sem).wait()

    def compute(c, yb, xb):
        idx_base = pt_base + c * _CHUNK + _PAD

        def vec_body(v, _):
            yv = yb[pl.ds(v * 16, 16)]
            xv = xb[pl.ds(v * 16, 16)]
            m = lax.shift_right_logical(yv, 6) == tid
            pix = (yv & (_B1_ROWS - 1)) * _W + xv
            pixc = jnp.where(m, pix, 0)
            plsc.store_scatter(wband, [pixc], zeros16, mask=m)
            plsc.addupdate_scatter(wband, [pixc], lane_bit, mask=m)
            got = plsc.load_gather(wband, [pixc], mask=m)
            keep = m & (got < lane_thr)
            val = (idx_base + v * 16) + iota
            plsc.store_scatter(wband, [pixc], val, mask=keep)
            return 0

        lax.fori_loop(0, _NVEC, vec_body, 0)

    issue(0, y0, x0, sem0)

    def pair_body(i, _):
        drain(y0, x0, sem0)
        issue(2 * i + 1, y1, x1, sem1)
        compute(2 * i, y0, x0)
        drain(y1, x1, sem1)

        @pl.when(2 * i + 2 < _NCHUNK)
        def _():
            issue(2 * i + 2, y0, x0, sem0)

        compute(2 * i + 1, y1, x1)
        return 0

    lax.fori_loop(0, _NCHUNK // 2, pair_body, 0)

    out_off = cid * (_H * _W) + tid * _B1_PIX
    pltpu.sync_copy(wband, w_hbm.at[pl.ds(out_off, _B1_PIX)])


def _merge_body(w_hbm, col_hbm, out_hbm, w0b, w1b, idxb, rows, sem):
    wid = lax.axis_index("s") * _NC + lax.axis_index("c")
    band_px = wid * _B2_PIX

    def win_body(g, _):
        px = band_px + g * _WIN
        pltpu.sync_copy(w_hbm.at[pl.ds(px, _WIN)], w0b)
        pltpu.sync_copy(w_hbm.at[pl.ds(_H * _W + px, _WIN)], w1b)

        def vec_body(v, _):
            a = w0b[pl.ds(v * 16, 16)]
            b = w1b[pl.ds(v * 16, 16)]
            m3 = jnp.maximum(a, b) * 3
            idxb[pl.ds(v * 16, 16)] = m3
            idxb[pl.ds(_WIN + v * 16, 16)] = m3 + 1
            idxb[pl.ds(2 * _WIN + v * 16, 16)] = m3 + 2
            return 0

        lax.fori_loop(0, _WIN // 16, vec_body, 0)
        for j in range(3 * _WIN // 128):
            pltpu.async_copy(col_hbm.at[idxb.at[pl.ds(j * 128, 128)]],
                             rows.at[pl.ds(j * 128, 128)], sem)
        for _ in range(3 * _WIN // 128):
            pltpu.make_async_copy(col_hbm.at[idxb.at[pl.ds(0, 128)]],
                                  rows.at[pl.ds(0, 128)], sem).wait()
        for ch in range(3):
            pltpu.sync_copy(rows.at[pl.ds(ch * _WIN, _WIN)],
                            out_hbm.at[pl.ds(ch * (_H * _W) + px, _WIN)])
        return 0

    lax.fori_loop(0, _NWIN, win_body, 0)


def _mesh():
    return plsc.VectorSubcoreMesh(core_axis_name="c", subcore_axis_name="s",
                                  num_cores=_NC, num_subcores=_NS)


@jax.jit
def _scan(y, x):
    mesh = _mesh()
    scan = pl.kernel(
        _scan_body,
        out_type=jax.ShapeDtypeStruct((_NC * _H * _W,), jnp.int32),
        mesh=mesh,
        compiler_params=pltpu.CompilerParams(needs_layout_passes=False),
        scratch_types=[
            pltpu.VMEM((_CHUNK,), jnp.int32),
            pltpu.VMEM((_CHUNK,), jnp.int32),
            pltpu.VMEM((_CHUNK,), jnp.int32),
            pltpu.VMEM((_CHUNK,), jnp.int32),
            pltpu.VMEM((_B1_PIX,), jnp.int32),
            pltpu.SemaphoreType.DMA,
            pltpu.SemaphoreType.DMA,
        ],
    )
    return scan(y, x)


@jax.jit
def _merge(w, colors_pad):
    mesh = _mesh()
    merge = pl.kernel(
        _merge_body,
        out_type=jax.ShapeDtypeStruct((3 * _H * _W,), jnp.float32),
        mesh=mesh,
        compiler_params=pltpu.CompilerParams(needs_layout_passes=False),
        scratch_types=[
            pltpu.VMEM((_WIN,), jnp.int32),
            pltpu.VMEM((_WIN,), jnp.int32),
            pltpu.VMEM((3 * _WIN,), jnp.int32),
            pltpu.VMEM((3 * _WIN,), jnp.float32),
            pltpu.SemaphoreType.DMA,
        ],
    )
    return merge(w, colors_pad)


def kernel(y, x, colors, height, width):
    zero_off = (jnp.asarray(height) - _H) + (jnp.asarray(width) - _W)
    y = (y + zero_off.astype(y.dtype)).astype(jnp.int32)
    x = (x + zero_off.astype(x.dtype)).astype(jnp.int32)
    colors_pad = jnp.concatenate(
        [jnp.zeros((_PAD * 3,), colors.dtype), colors.reshape(-1)])
    out = _merge(_scan(y, x), colors_pad)
    return out.reshape(3, _H, _W).transpose(1, 2, 0)


# speedup vs baseline: 20.4246x; 5.8115x over previous
import jax
import jax.numpy as jnp
import numpy as np
from jax import lax
from jax.experimental import pallas as pl
from jax.experimental.pallas import tpu as pltpu
from jax.experimental.pallas import tpu_sc as plsc

_H = 1024
_W = 1024
_N = 4_000_000
_PAD = 8

_NC = 2
_NS = 16
_NW = _NC * _NS

_HALF = _N // _NC
_B1_ROWS = _H // _NS
_B1_PIX = _B1_ROWS * _W
_CHUNK = 4000
_NCHUNK = _HALF // _CHUNK
_NVEC = _CHUNK // 16

_B2_ROWS = _H // _NW
_B2_PIX = _B2_ROWS * _W
_WIN = 2048
_NWIN = _B2_PIX // _WIN


def _scan_body(y_hbm, x_hbm, w_hbm, y0, x0, y1, x1, wband, sem0, sem1):
    cid = lax.axis_index("c")
    tid = lax.axis_index("s")
    iota = lax.iota(jnp.int32, 16)
    empty = iota & (_PAD - 1)
    lane_bit = lax.convert_element_type(
        lax.bitcast_convert_type(lax.shift_left(iota + 127, 23), jnp.float32),
        jnp.int32)
    lane_thr = lane_bit + lane_bit
    zeros16 = iota & 0
    pt_base = cid * _HALF

    def init_body(i, _):
        wband[pl.ds(i * 16, 16)] = empty
        return 0

    lax.fori_loop(0, _B1_PIX // 16, init_body, 0)

    def issue(c, yb, xb, sem):
        base = pt_base + c * _CHUNK
        pltpu.async_copy(y_hbm.at[pl.ds(base, _CHUNK)], yb, sem)
        pltpu.async_copy(x_hbm.at[pl.ds(base, _CHUNK)], xb, sem)

    def drain(yb, xb, sem):
        pltpu.make_async_copy(y_hbm.at[pl.ds(0, _CHUNK)], yb, sem).wait()
        pltpu.make_async_copy(x_hbm.at[pl.ds(0, _CHUNK)], xb, sem).wait()

    def compute(c, yb, xb):
        idx_base = pt_base + c * _CHUNK + _PAD

        def vec_body(v, _):
            yv = yb[pl.ds(v * 16, 16)]
            xv = xb[pl.ds(v * 16, 16)]
            m = lax.shift_right_logical(yv, 6) == tid
            pix = (yv & (_B1_ROWS - 1)) * _W + xv
            pixc = jnp.where(m, pix, 0)
            plsc.store_scatter(wband, [pixc], zeros16, mask=m)
            plsc.addupdate_scatter(wband, [pixc], lane_bit, mask=m)
            got = plsc.load_gather(wband, [pixc], mask=m)
            keep = m & (got < lane_thr)
            val = (idx_base + v * 16) + iota
            plsc.store_scatter(wband, [pixc], val, mask=keep)
            return 0

        lax.fori_loop(0, _NVEC, vec_body, 0)

    issue(0, y0, x0, sem0)

    def pair_body(i, _):
        drain(y0, x0, sem0)
        issue(2 * i + 1, y1, x1, sem1)
        compute(2 * i, y0, x0)
        drain(y1, x1, sem1)

        @pl.when(2 * i + 2 < _NCHUNK)
        def _():
            issue(2 * i + 2, y0, x0, sem0)

        compute(2 * i + 1, y1, x1)
        return 0

    lax.fori_loop(0, _NCHUNK // 2, pair_body, 0)

    out_off = cid * (_H * _W) + tid * _B1_PIX
    pltpu.sync_copy(wband, w_hbm.at[pl.ds(out_off, _B1_PIX)])


def _merge_body(w_hbm, col_hbm, out_hbm, w0b, w1b, idxb, rows, sem):
    wid = lax.axis_index("s") * _NC + lax.axis_index("c")
    band_px = wid * _B2_PIX

    def win_body(g, _):
        px = band_px + g * _WIN
        pltpu.sync_copy(w_hbm.at[pl.ds(px, _WIN)], w0b)
        pltpu.sync_copy(w_hbm.at[pl.ds(_H * _W + px, _WIN)], w1b)

        def vec_body(v, _):
            a = w0b[pl.ds(v * 16, 16)]
            b = w1b[pl.ds(v * 16, 16)]
            mx = jnp.maximum(a, b)
            idxb[pl.ds(v * 16, 16)] = mx
            idxb[pl.ds(_WIN + v * 16, 16)] = mx + (_N + _PAD)
            idxb[pl.ds(2 * _WIN + v * 16, 16)] = mx + 2 * (_N + _PAD)
            return 0

        lax.fori_loop(0, _WIN // 16, vec_body, 0)
        for j in range(3 * _WIN // 128):
            pltpu.async_copy(col_hbm.at[idxb.at[pl.ds(j * 128, 128)]],
                             rows.at[pl.ds(j * 128, 128)], sem)
        for _ in range(3 * _WIN // 128):
            pltpu.make_async_copy(col_hbm.at[idxb.at[pl.ds(0, 128)]],
                                  rows.at[pl.ds(0, 128)], sem).wait()
        for ch in range(3):
            pltpu.sync_copy(rows.at[pl.ds(ch * _WIN, _WIN)],
                            out_hbm.at[pl.ds(ch * (_H * _W) + px, _WIN)])
        return 0

    lax.fori_loop(0, _NWIN, win_body, 0)


def _mesh():
    return plsc.VectorSubcoreMesh(core_axis_name="c", subcore_axis_name="s",
                                  num_cores=_NC, num_subcores=_NS)


@jax.jit
def _scan(y, x):
    mesh = _mesh()
    scan = pl.kernel(
        _scan_body,
        out_type=jax.ShapeDtypeStruct((_NC * _H * _W,), jnp.int32),
        mesh=mesh,
        compiler_params=pltpu.CompilerParams(needs_layout_passes=False),
        scratch_types=[
            pltpu.VMEM((_CHUNK,), jnp.int32),
            pltpu.VMEM((_CHUNK,), jnp.int32),
            pltpu.VMEM((_CHUNK,), jnp.int32),
            pltpu.VMEM((_CHUNK,), jnp.int32),
            pltpu.VMEM((_B1_PIX,), jnp.int32),
            pltpu.SemaphoreType.DMA,
            pltpu.SemaphoreType.DMA,
        ],
    )
    return scan(y, x)


@jax.jit
def _merge(w, colors_pad):
    mesh = _mesh()
    merge = pl.kernel(
        _merge_body,
        out_type=jax.ShapeDtypeStruct((3 * _H * _W,), jnp.float32),
        mesh=mesh,
        compiler_params=pltpu.CompilerParams(needs_layout_passes=False),
        scratch_types=[
            pltpu.VMEM((_WIN,), jnp.int32),
            pltpu.VMEM((_WIN,), jnp.int32),
            pltpu.VMEM((3 * _WIN,), jnp.int32),
            pltpu.VMEM((3 * _WIN,), jnp.float32),
            pltpu.SemaphoreType.DMA,
        ],
    )
    return merge(w, colors_pad)


def kernel(y, x, colors, height, width):
    zero_off = (jnp.asarray(height) - _H) + (jnp.asarray(width) - _W)
    y = (y + zero_off.astype(y.dtype)).astype(jnp.int32)
    x = (x + zero_off.astype(x.dtype)).astype(jnp.int32)
    colors_pad = jnp.pad(colors, ((_PAD, 0), (0, 0))).T.reshape(-1)
    out = _merge(_scan(y, x), colors_pad)
    return out.reshape(3, _H, _W).transpose(1, 2, 0)


# speedup vs baseline: 21.2297x; 1.0394x over previous
import jax
import jax.numpy as jnp
import numpy as np
from jax import lax
from jax.experimental import pallas as pl
from jax.experimental.pallas import tpu as pltpu
from jax.experimental.pallas import tpu_sc as plsc

_H = 1024
_W = 1024
_N = 4_000_000
_PAD = 8

_NC = 2
_NS = 16
_NW = _NC * _NS

_HALF = _N // _NC
_B1_ROWS = _H // _NS
_B1_PIX = _B1_ROWS * _W
_CHUNK = 8000
_NCHUNK = _HALF // _CHUNK
_NVEC = _CHUNK // 16

_B2_ROWS = _H // _NW
_B2_PIX = _B2_ROWS * _W
_WIN = 2048
_NWIN = _B2_PIX // _WIN


def _scan_body(y_hbm, x_hbm, w_hbm, y0, x0, y1, x1, wband, sem0, sem1):
    cid = lax.axis_index("c")
    tid = lax.axis_index("s")
    iota = lax.iota(jnp.int32, 16)
    empty = iota & (_PAD - 1)
    lane_bit = lax.convert_element_type(
        lax.bitcast_convert_type(lax.shift_left(iota + 127, 23), jnp.float32),
        jnp.int32)
    lane_thr = lane_bit + lane_bit
    zeros16 = iota & 0
    pt_base = cid * _HALF

    def init_body(i, _):
        wband[pl.ds(i * 16, 16)] = empty
        return 0

    lax.fori_loop(0, _B1_PIX // 16, init_body, 0)

    def issue(c, yb, xb, sem):
        base = pt_base + c * _CHUNK
        pltpu.async_copy(y_hbm.at[pl.ds(base, _CHUNK)], yb, sem)
        pltpu.async_copy(x_hbm.at[pl.ds(base, _CHUNK)], xb, sem)

    def drain(yb, xb, sem):
        pltpu.make_async_copy(y_hbm.at[pl.ds(0, _CHUNK)], yb, sem).wait()
        pltpu.make_async_copy(x_hbm.at[pl.ds(0, _CHUNK)], xb, sem).wait()

    def compute(c, yb, xb):
        idx_base = pt_base + c * _CHUNK + _PAD

        def one_vec(v):
            yv = yb[pl.ds(v * 16, 16)]
            xv = xb[pl.ds(v * 16, 16)]
            m = lax.shift_right_logical(yv, 6) == tid
            pix = (yv & (_B1_ROWS - 1)) * _W + xv
            pixc = jnp.where(m, pix, 0)
            plsc.store_scatter(wband, [pixc], zeros16, mask=m)
            plsc.addupdate_scatter(wband, [pixc], lane_bit, mask=m)
            got = plsc.load_gather(wband, [pixc], mask=m)
            keep = m & (got < lane_thr)
            val = (idx_base + v * 16) + iota
            plsc.store_scatter(wband, [pixc], val, mask=keep)

        def vec_body(i, _):
            one_vec(2 * i)
            one_vec(2 * i + 1)
            return 0

        lax.fori_loop(0, _NVEC // 2, vec_body, 0)

    issue(0, y0, x0, sem0)

    def pair_body(i, _):
        drain(y0, x0, sem0)
        issue(2 * i + 1, y1, x1, sem1)
        compute(2 * i, y0, x0)
        drain(y1, x1, sem1)

        @pl.when(2 * i + 2 < _NCHUNK)
        def _():
            issue(2 * i + 2, y0, x0, sem0)

        compute(2 * i + 1, y1, x1)
        return 0

    lax.fori_loop(0, _NCHUNK // 2, pair_body, 0)

    out_off = cid * (_H * _W) + tid * _B1_PIX
    pltpu.sync_copy(wband, w_hbm.at[pl.ds(out_off, _B1_PIX)])


def _merge_body(w_hbm, col_hbm, out_hbm, w0b, w1b, idxb, rows, sem):
    wid = lax.axis_index("s") * _NC + lax.axis_index("c")
    band_px = wid * _B2_PIX

    def win_body(g, _):
        px = band_px + g * _WIN
        pltpu.sync_copy(w_hbm.at[pl.ds(px, _WIN)], w0b)
        pltpu.sync_copy(w_hbm.at[pl.ds(_H * _W + px, _WIN)], w1b)

        def vec_body(v, _):
            a = w0b[pl.ds(v * 16, 16)]
            b = w1b[pl.ds(v * 16, 16)]
            mx = jnp.maximum(a, b)
            idxb[pl.ds(v * 16, 16)] = mx
            idxb[pl.ds(_WIN + v * 16, 16)] = mx + (_N + _PAD)
            idxb[pl.ds(2 * _WIN + v * 16, 16)] = mx + 2 * (_N + _PAD)
            return 0

        lax.fori_loop(0, _WIN // 16, vec_body, 0)
        for j in range(3 * _WIN // 128):
            pltpu.async_copy(col_hbm.at[idxb.at[pl.ds(j * 128, 128)]],
                             rows.at[pl.ds(j * 128, 128)], sem)
        for _ in range(3 * _WIN // 128):
            pltpu.make_async_copy(col_hbm.at[idxb.at[pl.ds(0, 128)]],
                                  rows.at[pl.ds(0, 128)], sem).wait()
        for ch in range(3):
            pltpu.sync_copy(rows.at[pl.ds(ch * _WIN, _WIN)],
                            out_hbm.at[pl.ds(ch * (_H * _W) + px, _WIN)])
        return 0

    lax.fori_loop(0, _NWIN, win_body, 0)


def _mesh():
    return plsc.VectorSubcoreMesh(core_axis_name="c", subcore_axis_name="s",
                                  num_cores=_NC, num_subcores=_NS)


@jax.jit
def _scan(y, x):
    mesh = _mesh()
    scan = pl.kernel(
        _scan_body,
        out_type=jax.ShapeDtypeStruct((_NC * _H * _W,), jnp.int32),
        mesh=mesh,
        compiler_params=pltpu.CompilerParams(needs_layout_passes=False),
        scratch_types=[
            pltpu.VMEM((_CHUNK,), jnp.int32),
            pltpu.VMEM((_CHUNK,), jnp.int32),
            pltpu.VMEM((_CHUNK,), jnp.int32),
            pltpu.VMEM((_CHUNK,), jnp.int32),
            pltpu.VMEM((_B1_PIX,), jnp.int32),
            pltpu.SemaphoreType.DMA,
            pltpu.SemaphoreType.DMA,
        ],
    )
    return scan(y, x)


@jax.jit
def _merge(w, colors_pad):
    mesh = _mesh()
    merge = pl.kernel(
        _merge_body,
        out_type=jax.ShapeDtypeStruct((3 * _H * _W,), jnp.float32),
        mesh=mesh,
        compiler_params=pltpu.CompilerParams(needs_layout_passes=False),
        scratch_types=[
            pltpu.VMEM((_WIN,), jnp.int32),
            pltpu.VMEM((_WIN,), jnp.int32),
            pltpu.VMEM((3 * _WIN,), jnp.int32),
            pltpu.VMEM((3 * _WIN,), jnp.float32),
            pltpu.SemaphoreType.DMA,
        ],
    )
    return merge(w, colors_pad)


def kernel(y, x, colors, height, width):
    zero_off = (jnp.asarray(height) - _H) + (jnp.asarray(width) - _W)
    y = (y + zero_off.astype(y.dtype)).astype(jnp.int32)
    x = (x + zero_off.astype(x.dtype)).astype(jnp.int32)
    colors_pad = jnp.pad(colors, ((_PAD, 0), (0, 0))).T.reshape(-1)
    out = _merge(_scan(y, x), colors_pad)
    return out.reshape(3, _H, _W).transpose(1, 2, 0)


# speedup vs baseline: 21.7214x; 1.0232x over previous
import jax
import jax.numpy as jnp
import numpy as np
from jax import lax
from jax.experimental import pallas as pl
from jax.experimental.pallas import tpu as pltpu
from jax.experimental.pallas import tpu_sc as plsc

_H = 1024
_W = 1024
_N = 4_000_000
_PAD = 8

_NC = 2
_NS = 16
_NW = _NC * _NS

_HALF = _N // _NC
_B1_ROWS = _H // _NS
_B1_PIX = _B1_ROWS * _W
_CHUNK = 8000
_NCHUNK = _HALF // _CHUNK
_NVEC = _CHUNK // 16

_B2_ROWS = _H // _NW
_B2_PIX = _B2_ROWS * _W
_WIN = 2048
_NWIN = _B2_PIX // _WIN


def _scan_body(y_hbm, x_hbm, w_hbm, y0, x0, y1, x1, wband, sem0, sem1):
    cid = lax.axis_index("c")
    tid = lax.axis_index("s")
    iota = lax.iota(jnp.int32, 16)
    empty = iota & (_PAD - 1)
    lane_bit = lax.convert_element_type(
        lax.bitcast_convert_type(lax.shift_left(iota + 127, 23), jnp.float32),
        jnp.int32)
    lane_thr = lane_bit + lane_bit
    zeros16 = iota & 0
    pt_base = cid * _HALF

    def init_body(i, _):
        wband[pl.ds(i * 16, 16)] = empty
        return 0

    lax.fori_loop(0, _B1_PIX // 16, init_body, 0)

    def issue(c, yb, xb, sem):
        base = pt_base + c * _CHUNK
        pltpu.async_copy(y_hbm.at[pl.ds(base, _CHUNK)], yb, sem)
        pltpu.async_copy(x_hbm.at[pl.ds(base, _CHUNK)], xb, sem)

    def drain(yb, xb, sem):
        pltpu.make_async_copy(y_hbm.at[pl.ds(0, _CHUNK)], yb, sem).wait()
        pltpu.make_async_copy(x_hbm.at[pl.ds(0, _CHUNK)], xb, sem).wait()

    def compute(c, yb, xb):
        idx_base = pt_base + c * _CHUNK + _PAD

        def one_vec(v):
            yv = yb[pl.ds(v * 16, 16)]
            xv = xb[pl.ds(v * 16, 16)]
            m = lax.shift_right_logical(yv, 6) == tid
            pix = (yv & (_B1_ROWS - 1)) * _W + xv
            pixc = jnp.where(m, pix, 0)
            plsc.store_scatter(wband, [pixc], zeros16, mask=m)
            plsc.addupdate_scatter(wband, [pixc], lane_bit, mask=m)
            got = plsc.load_gather(wband, [pixc], mask=m)
            keep = m & (got < lane_thr)
            val = (idx_base + v * 16) + iota
            plsc.store_scatter(wband, [pixc], val, mask=keep)

        def vec_body(i, _):
            one_vec(4 * i)
            one_vec(4 * i + 1)
            one_vec(4 * i + 2)
            one_vec(4 * i + 3)
            return 0

        lax.fori_loop(0, _NVEC // 4, vec_body, 0)

    issue(0, y0, x0, sem0)

    def pair_body(i, _):
        drain(y0, x0, sem0)
        issue(2 * i + 1, y1, x1, sem1)
        compute(2 * i, y0, x0)
        drain(y1, x1, sem1)

        @pl.when(2 * i + 2 < _NCHUNK)
        def _():
            issue(2 * i + 2, y0, x0, sem0)

        compute(2 * i + 1, y1, x1)
        return 0

    lax.fori_loop(0, _NCHUNK // 2, pair_body, 0)

    out_off = cid * (_H * _W) + tid * _B1_PIX
    pltpu.sync_copy(wband, w_hbm.at[pl.ds(out_off, _B1_PIX)])


def _merge_body(w_hbm, col_hbm, out_hbm, w0b, w1b, idxb, rows, sem):
    wid = lax.axis_index("s") * _NC + lax.axis_index("c")
    band_px = wid * _B2_PIX

    def win_body(g, _):
        px = band_px + g * _WIN
        pltpu.sync_copy(w_hbm.at[pl.ds(px, _WIN)], w0b)
        pltpu.sync_copy(w_hbm.at[pl.ds(_H * _W + px, _WIN)], w1b)

        def vec_body(v, _):
            a = w0b[pl.ds(v * 16, 16)]
            b = w1b[pl.ds(v * 16, 16)]
            mx = jnp.maximum(a, b)
            idxb[pl.ds(v * 16, 16)] = mx
            idxb[pl.ds(_WIN + v * 16, 16)] = mx + (_N + _PAD)
            idxb[pl.ds(2 * _WIN + v * 16, 16)] = mx + 2 * (_N + _PAD)
            return 0

        lax.fori_loop(0, _WIN // 16, vec_body, 0)
        for j in range(3 * _WIN // 128):
            pltpu.async_copy(col_hbm.at[idxb.at[pl.ds(j * 128, 128)]],
                             rows.at[pl.ds(j * 128, 128)], sem)
        for _ in range(3 * _WIN // 128):
            pltpu.make_async_copy(col_hbm.at[idxb.at[pl.ds(0, 128)]],
                                  rows.at[pl.ds(0, 128)], sem).wait()
        for ch in range(3):
            pltpu.sync_copy(rows.at[pl.ds(ch * _WIN, _WIN)],
                            out_hbm.at[pl.ds(ch * (_H * _W) + px, _WIN)])
        return 0

    lax.fori_loop(0, _NWIN, win_body, 0)


def _mesh():
    return plsc.VectorSubcoreMesh(core_axis_name="c", subcore_axis_name="s",
                                  num_cores=_NC, num_subcores=_NS)


@jax.jit
def _scan(y, x):
    mesh = _mesh()
    scan = pl.kernel(
        _scan_body,
        out_type=jax.ShapeDtypeStruct((_NC * _H * _W,), jnp.int32),
        mesh=mesh,
        compiler_params=pltpu.CompilerParams(needs_layout_passes=False),
        scratch_types=[
            pltpu.VMEM((_CHUNK,), jnp.int32),
            pltpu.VMEM((_CHUNK,), jnp.int32),
            pltpu.VMEM((_CHUNK,), jnp.int32),
            pltpu.VMEM((_CHUNK,), jnp.int32),
            pltpu.VMEM((_B1_PIX,), jnp.int32),
            pltpu.SemaphoreType.DMA,
            pltpu.SemaphoreType.DMA,
        ],
    )
    return scan(y, x)


@jax.jit
def _merge(w, colors_pad):
    mesh = _mesh()
    merge = pl.kernel(
        _merge_body,
        out_type=jax.ShapeDtypeStruct((3 * _H * _W,), jnp.float32),
        mesh=mesh,
        compiler_params=pltpu.CompilerParams(needs_layout_passes=False),
        scratch_types=[
            pltpu.VMEM((_WIN,), jnp.int32),
            pltpu.VMEM((_WIN,), jnp.int32),
            pltpu.VMEM((3 * _WIN,), jnp.int32),
            pltpu.VMEM((3 * _WIN,), jnp.float32),
            pltpu.SemaphoreType.DMA,
        ],
    )
    return merge(w, colors_pad)


def kernel(y, x, colors, height, width):
    zero_off = (jnp.asarray(height) - _H) + (jnp.asarray(width) - _W)
    y = (y + zero_off.astype(y.dtype)).astype(jnp.int32)
    x = (x + zero_off.astype(x.dtype)).astype(jnp.int32)
    colors_pad = jnp.pad(colors, ((_PAD, 0), (0, 0))).T.reshape(-1)
    out = _merge(_scan(y, x), colors_pad)
    return out.reshape(3, _H, _W).transpose(1, 2, 0)
